# baseline (speedup 1.0000x reference)
import dataclasses
import functools

import jax
import jax.numpy as jnp
from jax import lax
from jax.experimental import pallas as pl
from jax.experimental.pallas import tpu as pltpu
from jax.experimental.pallas import tpu_sc as plsc

NUM_SA = 100000
FEATURE_DIM = 128
BATCH = 16384

NC = 2
NS = 16
L = 16
NW = NC * NS
B_PER_W = BATCH // NW
GATHER_W = 128
N_CHUNKS = B_PER_W // GATHER_W


def _make_kernel():
    mesh = plsc.VectorSubcoreMesh(core_axis_name="c", subcore_axis_name="s")
    cp = pltpu.CompilerParams()
    if "needs_layout_passes" in pltpu.CompilerParams.__dataclass_fields__:
        cp = dataclasses.replace(cp, needs_layout_passes=False)

    @functools.partial(
        pl.kernel,
        mesh=mesh,
        compiler_params=cp,
        out_type=jax.ShapeDtypeStruct((BATCH,), jnp.float32),
        scratch_types=[
            pltpu.VMEM((B_PER_W,), jnp.int32),
            pltpu.VMEM((B_PER_W,), jnp.int32),
            pltpu.VMEM((B_PER_W,), jnp.int32),
            pltpu.VMEM((FEATURE_DIM,), jnp.float32),
            pltpu.VMEM((B_PER_W, FEATURE_DIM), jnp.float32),
            pltpu.VMEM((B_PER_W,), jnp.float32),
            pltpu.SemaphoreType.DMA,
            pltpu.SemaphoreType.DMA((N_CHUNKS,)),
        ],
    )
    def k(states_hbm, actions_hbm, table_hbm, w_hbm, out_hbm,
          st_v, ac_v, idx_v, w_v, rows_v, out_v, sem_in, sem_g):
        wid = lax.axis_index("s") * NC + lax.axis_index("c")
        base = wid * B_PER_W

        cp_st = pltpu.async_copy(states_hbm.at[pl.ds(base, B_PER_W)], st_v, sem_in)
        cp_ac = pltpu.async_copy(actions_hbm.at[pl.ds(base, B_PER_W)], ac_v, sem_in)
        cp_w = pltpu.async_copy(w_hbm, w_v, sem_in)
        cp_st.wait()
        cp_ac.wait()

        @plsc.parallel_loop(0, B_PER_W, step=L, unroll=1)
        def _(j):
            sl = pl.ds(j, L)
            idx_v[sl] = st_v[sl] * 4 + ac_v[sl]

        for c in range(N_CHUNKS):
            pltpu.async_copy(
                table_hbm.at[idx_v.at[pl.ds(c * GATHER_W, GATHER_W)]],
                rows_v.at[pl.ds(c * GATHER_W, GATHER_W)],
                sem_g.at[c],
            )

        cp_w.wait()
        w_regs = [w_v[pl.ds(j * L, L)] for j in range(FEATURE_DIM // L)]
        lane = lax.iota(jnp.int32, L)

        @pl.loop(0, N_CHUNKS)
        def _(c):
            pltpu.make_async_copy(
                table_hbm.at[pl.ds(0, GATHER_W)],
                rows_v.at[pl.ds(c * GATHER_W, GATHER_W)],
                sem_g.at[c],
            ).wait()

            @plsc.parallel_loop(0, GATHER_W, step=L, unroll=1)
            def _(jj):
                i0 = c * GATHER_W + jj
                res = jnp.zeros((L,), jnp.float32)
                for rr in range(L):
                    i = i0 + rr
                    acc = rows_v[i, pl.ds(0, L)] * w_regs[0]
                    for j in range(1, FEATURE_DIM // L):
                        acc += rows_v[i, pl.ds(j * L, L)] * w_regs[j]
                    res = jnp.where(lane == rr, jnp.sum(acc), res)
                out_v[pl.ds(i0, L)] = res

        pltpu.sync_copy(out_v, out_hbm.at[pl.ds(base, B_PER_W)])

    return k


_kernel_cache = []


@jax.jit
def _run(states, actions, layer1, W):
    if not _kernel_cache:
        _kernel_cache.append(_make_kernel())
    return _kernel_cache[0](states.astype(jnp.int32), actions.astype(jnp.int32),
                            layer1, W.reshape(FEATURE_DIM))


def kernel(states, actions, layer1, W):
    r = _run(states, actions, layer1, W)
    return (r, 0)


# speedup vs baseline: 1.0522x; 1.0522x over previous
import dataclasses
import functools

import jax
import jax.numpy as jnp
from jax import lax
from jax.experimental import pallas as pl
from jax.experimental.pallas import tpu as pltpu
from jax.experimental.pallas import tpu_sc as plsc

NUM_SA = 100000
FEATURE_DIM = 128
BATCH = 16384

NC = 2
NS = 16
L = 16
NW = NC * NS
B_PER_W = BATCH // NW
GATHER_W = 64
N_CHUNKS = B_PER_W // GATHER_W


def _make_kernel():
    mesh = plsc.VectorSubcoreMesh(core_axis_name="c", subcore_axis_name="s")
    cp = pltpu.CompilerParams()
    if "needs_layout_passes" in pltpu.CompilerParams.__dataclass_fields__:
        cp = dataclasses.replace(cp, needs_layout_passes=False)

    @functools.partial(
        pl.kernel,
        mesh=mesh,
        compiler_params=cp,
        out_type=jax.ShapeDtypeStruct((BATCH,), jnp.float32),
        scratch_types=[
            pltpu.VMEM((B_PER_W,), jnp.int32),
            pltpu.VMEM((B_PER_W,), jnp.int32),
            pltpu.VMEM((B_PER_W,), jnp.int32),
            pltpu.VMEM((FEATURE_DIM,), jnp.float32),
            pltpu.VMEM((B_PER_W, FEATURE_DIM), jnp.float32),
            pltpu.VMEM((B_PER_W,), jnp.float32),
            pltpu.SemaphoreType.DMA,
            pltpu.SemaphoreType.DMA((N_CHUNKS,)),
        ],
    )
    def k(states_hbm, actions_hbm, table_hbm, w_hbm, out_hbm,
          st_v, ac_v, idx_v, w_v, rows_v, out_v, sem_in, sem_g):
        wid = lax.axis_index("s") * NC + lax.axis_index("c")
        base = wid * B_PER_W

        cp_st = pltpu.async_copy(states_hbm.at[pl.ds(base, B_PER_W)], st_v, sem_in)
        cp_ac = pltpu.async_copy(actions_hbm.at[pl.ds(base, B_PER_W)], ac_v, sem_in)
        cp_w = pltpu.async_copy(w_hbm, w_v, sem_in)
        cp_st.wait()
        cp_ac.wait()

        @pl.loop(0, B_PER_W, step=L)
        def _(j):
            sl = pl.ds(j, L)
            idx_v[sl] = st_v[sl] * 4 + ac_v[sl]

        for c in range(N_CHUNKS):
            pltpu.async_copy(
                table_hbm.at[idx_v.at[pl.ds(c * GATHER_W, GATHER_W)]],
                rows_v.at[pl.ds(c * GATHER_W, GATHER_W)],
                sem_g.at[c],
            )

        cp_w.wait()
        w_regs = [w_v[pl.ds(j * L, L)] for j in range(FEATURE_DIM // L)]
        lane = lax.iota(jnp.int32, L)

        @pl.loop(0, N_CHUNKS)
        def _(c):
            pltpu.make_async_copy(
                table_hbm.at[pl.ds(0, GATHER_W)],
                rows_v.at[pl.ds(c * GATHER_W, GATHER_W)],
                sem_g.at[c],
            ).wait()

            @pl.loop(0, GATHER_W, step=L)
            def _(jj):
                i0 = c * GATHER_W + jj
                res = jnp.zeros((L,), jnp.float32)
                for rr in range(L):
                    i = i0 + rr
                    acc = rows_v[i, pl.ds(0, L)] * w_regs[0]
                    for j in range(1, FEATURE_DIM // L):
                        acc += rows_v[i, pl.ds(j * L, L)] * w_regs[j]
                    res = jnp.where(lane == rr, jnp.sum(acc), res)
                out_v[pl.ds(i0, L)] = res

        pltpu.sync_copy(out_v, out_hbm.at[pl.ds(base, B_PER_W)])

    return k


_kernel_cache = []


@jax.jit
def _run(states, actions, layer1, W):
    if not _kernel_cache:
        _kernel_cache.append(_make_kernel())
    return _kernel_cache[0](states.astype(jnp.int32), actions.astype(jnp.int32),
                            layer1, W.reshape(FEATURE_DIM))


def kernel(states, actions, layer1, W):
    r = _run(states, actions, layer1, W)
    return (r, 0)


# speedup vs baseline: 1.0912x; 1.0371x over previous
import dataclasses
import functools

import jax
import jax.numpy as jnp
from jax import lax
from jax.experimental import pallas as pl
from jax.experimental.pallas import tpu as pltpu
from jax.experimental.pallas import tpu_sc as plsc

NUM_SA = 100000
FEATURE_DIM = 128
BATCH = 16384

NC = 2
NS = 16
L = 16
NW = NC * NS
B_PER_W = BATCH // NW
GATHER_W = 32
N_CHUNKS = B_PER_W // GATHER_W


def _make_kernel():
    mesh = plsc.VectorSubcoreMesh(core_axis_name="c", subcore_axis_name="s")
    cp = pltpu.CompilerParams()
    if "needs_layout_passes" in pltpu.CompilerParams.__dataclass_fields__:
        cp = dataclasses.replace(cp, needs_layout_passes=False)

    @functools.partial(
        pl.kernel,
        mesh=mesh,
        compiler_params=cp,
        out_type=jax.ShapeDtypeStruct((BATCH,), jnp.float32),
        scratch_types=[
            pltpu.VMEM((B_PER_W,), jnp.int32),
            pltpu.VMEM((B_PER_W,), jnp.int32),
            pltpu.VMEM((B_PER_W,), jnp.int32),
            pltpu.VMEM((FEATURE_DIM,), jnp.float32),
            pltpu.VMEM((B_PER_W, FEATURE_DIM), jnp.float32),
            pltpu.VMEM((B_PER_W,), jnp.float32),
            pltpu.SemaphoreType.DMA,
            pltpu.SemaphoreType.DMA((N_CHUNKS,)),
        ],
    )
    def k(states_hbm, actions_hbm, table_hbm, w_hbm, out_hbm,
          st_v, ac_v, idx_v, w_v, rows_v, out_v, sem_in, sem_g):
        wid = lax.axis_index("s") * NC + lax.axis_index("c")
        base = wid * B_PER_W

        cp_st = pltpu.async_copy(states_hbm.at[pl.ds(base, B_PER_W)], st_v, sem_in)
        cp_ac = pltpu.async_copy(actions_hbm.at[pl.ds(base, B_PER_W)], ac_v, sem_in)
        cp_w = pltpu.async_copy(w_hbm, w_v, sem_in)
        cp_st.wait()
        cp_ac.wait()

        @pl.loop(0, B_PER_W, step=L)
        def _(j):
            sl = pl.ds(j, L)
            idx_v[sl] = st_v[sl] * 4 + ac_v[sl]

        for c in range(N_CHUNKS):
            pltpu.async_copy(
                table_hbm.at[idx_v.at[pl.ds(c * GATHER_W, GATHER_W)]],
                rows_v.at[pl.ds(c * GATHER_W, GATHER_W)],
                sem_g.at[c],
            )

        cp_w.wait()
        w_regs = [w_v[pl.ds(j * L, L)] for j in range(FEATURE_DIM // L)]
        lane = lax.iota(jnp.int32, L)

        @pl.loop(0, N_CHUNKS)
        def _(c):
            pltpu.make_async_copy(
                table_hbm.at[pl.ds(0, GATHER_W)],
                rows_v.at[pl.ds(c * GATHER_W, GATHER_W)],
                sem_g.at[c],
            ).wait()

            @pl.loop(0, GATHER_W, step=L)
            def _(jj):
                i0 = c * GATHER_W + jj
                res = jnp.zeros((L,), jnp.float32)
                for rr in range(L):
                    i = i0 + rr
                    acc = rows_v[i, pl.ds(0, L)] * w_regs[0]
                    for j in range(1, FEATURE_DIM // L):
                        acc += rows_v[i, pl.ds(j * L, L)] * w_regs[j]
                    res = jnp.where(lane == rr, jnp.sum(acc), res)
                out_v[pl.ds(i0, L)] = res

        pltpu.sync_copy(out_v, out_hbm.at[pl.ds(base, B_PER_W)])

    return k


_kernel_cache = []


@jax.jit
def _run(states, actions, layer1, W):
    if not _kernel_cache:
        _kernel_cache.append(_make_kernel())
    return _kernel_cache[0](states.astype(jnp.int32), actions.astype(jnp.int32),
                            layer1, W.reshape(FEATURE_DIM))


def kernel(states, actions, layer1, W):
    r = _run(states, actions, layer1, W)
    return (r, 0)


# speedup vs baseline: 1.1046x; 1.0122x over previous
import dataclasses
import functools

import jax
import jax.numpy as jnp
from jax import lax
from jax.experimental import pallas as pl
from jax.experimental.pallas import tpu as pltpu
from jax.experimental.pallas import tpu_sc as plsc

NUM_SA = 100000
FEATURE_DIM = 128
BATCH = 16384

NC = 2
NS = 16
L = 16
NW = NC * NS
B_PER_W = BATCH // NW
GATHER_W = 32
N_CHUNKS = B_PER_W // GATHER_W


def _make_kernel():
    mesh = plsc.VectorSubcoreMesh(core_axis_name="c", subcore_axis_name="s")
    cp = pltpu.CompilerParams()
    if "needs_layout_passes" in pltpu.CompilerParams.__dataclass_fields__:
        cp = dataclasses.replace(cp, needs_layout_passes=False)

    @functools.partial(
        pl.kernel,
        mesh=mesh,
        compiler_params=cp,
        out_type=jax.ShapeDtypeStruct((BATCH,), jnp.float32),
        scratch_types=[
            pltpu.VMEM((B_PER_W,), jnp.int32),
            pltpu.VMEM((B_PER_W,), jnp.int32),
            pltpu.VMEM((B_PER_W,), jnp.int32),
            pltpu.VMEM((FEATURE_DIM,), jnp.float32),
            pltpu.VMEM((B_PER_W, FEATURE_DIM), jnp.float32),
            pltpu.VMEM((B_PER_W,), jnp.float32),
            pltpu.SemaphoreType.DMA,
            pltpu.SemaphoreType.DMA((N_CHUNKS,)),
        ],
    )
    def k(states_hbm, actions_hbm, table_hbm, w_hbm, out_hbm,
          st_v, ac_v, idx_v, w_v, rows_v, out_v, sem_in, sem_g):
        wid = lax.axis_index("s") * NC + lax.axis_index("c")
        base = wid * B_PER_W

        cp_st = pltpu.async_copy(states_hbm.at[pl.ds(base, B_PER_W)], st_v, sem_in)
        cp_ac = pltpu.async_copy(actions_hbm.at[pl.ds(base, B_PER_W)], ac_v, sem_in)
        cp_w = pltpu.async_copy(w_hbm, w_v, sem_in)
        cp_st.wait()
        cp_ac.wait()

        @pl.loop(0, N_CHUNKS)
        def _(c):
            for jj in range(GATHER_W // L):
                sl = pl.ds(c * GATHER_W + jj * L, L)
                idx_v[sl] = st_v[sl] * 4 + ac_v[sl]
            pltpu.async_copy(
                table_hbm.at[idx_v.at[pl.ds(c * GATHER_W, GATHER_W)]],
                rows_v.at[pl.ds(c * GATHER_W, GATHER_W)],
                sem_g.at[c],
            )

        cp_w.wait()
        w_regs = [w_v[pl.ds(j * L, L)] for j in range(FEATURE_DIM // L)]
        lane = lax.iota(jnp.int32, L)

        @pl.loop(0, N_CHUNKS)
        def _(c):
            pltpu.make_async_copy(
                table_hbm.at[pl.ds(0, GATHER_W)],
                rows_v.at[pl.ds(c * GATHER_W, GATHER_W)],
                sem_g.at[c],
            ).wait()

            @pl.loop(0, GATHER_W, step=L)
            def _(jj):
                i0 = c * GATHER_W + jj
                res = jnp.zeros((L,), jnp.float32)
                for rr in range(L):
                    i = i0 + rr
                    acc = rows_v[i, pl.ds(0, L)] * w_regs[0]
                    for j in range(1, FEATURE_DIM // L):
                        acc += rows_v[i, pl.ds(j * L, L)] * w_regs[j]
                    res = jnp.where(lane == rr, jnp.sum(acc), res)
                out_v[pl.ds(i0, L)] = res

        pltpu.sync_copy(out_v, out_hbm.at[pl.ds(base, B_PER_W)])

    return k


_kernel_cache = []


@jax.jit
def _run(states, actions, layer1, W):
    if not _kernel_cache:
        _kernel_cache.append(_make_kernel())
    return _kernel_cache[0](states.astype(jnp.int32), actions.astype(jnp.int32),
                            layer1, W.reshape(FEATURE_DIM))


def kernel(states, actions, layer1, W):
    r = _run(states, actions, layer1, W)
    return (r, 0)


# speedup vs baseline: 1.1162x; 1.0105x over previous
import dataclasses
import functools

import jax
import jax.numpy as jnp
from jax import lax
from jax.experimental import pallas as pl
from jax.experimental.pallas import tpu as pltpu
from jax.experimental.pallas import tpu_sc as plsc

NUM_SA = 100000
FEATURE_DIM = 128
BATCH = 16384

NC = 2
NS = 16
L = 16
NW = NC * NS
B_PER_W = BATCH // NW
GATHER_W = 32
N_CHUNKS = B_PER_W // GATHER_W


def _make_kernel():
    mesh = plsc.VectorSubcoreMesh(core_axis_name="c", subcore_axis_name="s")
    cp = pltpu.CompilerParams()
    if "needs_layout_passes" in pltpu.CompilerParams.__dataclass_fields__:
        cp = dataclasses.replace(cp, needs_layout_passes=False)

    @functools.partial(
        pl.kernel,
        mesh=mesh,
        compiler_params=cp,
        out_type=jax.ShapeDtypeStruct((BATCH,), jnp.float32),
        scratch_types=[
            pltpu.VMEM((B_PER_W,), jnp.int32),
            pltpu.VMEM((B_PER_W,), jnp.int32),
            pltpu.VMEM((B_PER_W,), jnp.int32),
            pltpu.VMEM((FEATURE_DIM,), jnp.float32),
            pltpu.VMEM((B_PER_W, FEATURE_DIM), jnp.float32),
            pltpu.VMEM((B_PER_W,), jnp.float32),
            pltpu.SemaphoreType.DMA,
            pltpu.SemaphoreType.DMA((N_CHUNKS,)),
        ],
    )
    def k(states_hbm, actions_hbm, table_hbm, w_hbm, out_hbm,
          st_v, ac_v, idx_v, w_v, rows_v, out_v, sem_in, sem_g):
        wid = lax.axis_index("s") * NC + lax.axis_index("c")
        base = wid * B_PER_W

        cp_st = pltpu.async_copy(states_hbm.at[pl.ds(base, B_PER_W)], st_v, sem_in)
        cp_ac = pltpu.async_copy(actions_hbm.at[pl.ds(base, B_PER_W)], ac_v, sem_in)
        cp_w = pltpu.async_copy(w_hbm, w_v, sem_in)
        cp_st.wait()
        cp_ac.wait()

        @pl.loop(0, N_CHUNKS)
        def _(c):
            for jj in range(GATHER_W // L):
                sl = pl.ds(c * GATHER_W + jj * L, L)
                idx_v[sl] = st_v[sl] * 4 + ac_v[sl]
            pltpu.async_copy(
                table_hbm.at[idx_v.at[pl.ds(c * GATHER_W, GATHER_W)]],
                rows_v.at[pl.ds(c * GATHER_W, GATHER_W)],
                sem_g.at[c],
            )

        cp_w.wait()
        w_regs = [w_v[pl.ds(j * L, L)] for j in range(FEATURE_DIM // L)]
        lane = lax.iota(jnp.int32, L)

        @pl.loop(0, N_CHUNKS)
        def _(c):
            pltpu.make_async_copy(
                table_hbm.at[pl.ds(0, GATHER_W)],
                rows_v.at[pl.ds(c * GATHER_W, GATHER_W)],
                sem_g.at[c],
            ).wait()

            @pl.loop(0, GATHER_W, step=L)
            def _(jj):
                i0 = c * GATHER_W + jj

                def half(h, res):
                    for rr in range(L // 2):
                        i = i0 + h * (L // 2) + rr
                        acc = rows_v[i, pl.ds(0, L)] * w_regs[0]
                        for j in range(1, FEATURE_DIM // L):
                            acc += rows_v[i, pl.ds(j * L, L)] * w_regs[j]
                        res = jnp.where(lane == h * (L // 2) + rr,
                                        jnp.sum(acc), res)
                    return res

                res = lax.fori_loop(0, 2, half, jnp.zeros((L,), jnp.float32))
                out_v[pl.ds(i0, L)] = res

        pltpu.sync_copy(out_v, out_hbm.at[pl.ds(base, B_PER_W)])

    return k


_kernel_cache = []


@jax.jit
def _run(states, actions, layer1, W):
    if not _kernel_cache:
        _kernel_cache.append(_make_kernel())
    return _kernel_cache[0](states.astype(jnp.int32), actions.astype(jnp.int32),
                            layer1, W.reshape(FEATURE_DIM))


def kernel(states, actions, layer1, W):
    r = _run(states, actions, layer1, W)
    return (r, 0)
